# baseline (speedup 1.0000x reference)
import functools

import jax
import jax.numpy as jnp
from jax import lax
from jax.experimental import pallas as pl
from jax.experimental.pallas import tpu as pltpu
from jax.experimental.pallas import tpu_sc as plsc

N = 10000
E = 320000
D = 128

CHUNK = 128
NUM_CHUNKS = E // CHUNK
NC = 2
NS = 16
NW = NC * NS
CPW = NUM_CHUNKS // NW
EXTRA = NUM_CHUNKS - CPW * NW
MAXC = CPW + 1
OUTER = (MAXC + 2) // 3

ROWS_PER_TILE = 624
REM0 = NS * ROWS_PER_TILE


def _sc_aggregate(x, src, dst, zeros):
    mesh = plsc.VectorSubcoreMesh(core_axis_name="c", subcore_axis_name="s")

    @functools.partial(
        pl.kernel,
        mesh=mesh,
        out_type=jax.ShapeDtypeStruct((NC, N, D), jnp.float32),
        scratch_types=[
            pltpu.VMEM((CHUNK,), jnp.int32),
            pltpu.VMEM((CHUNK,), jnp.int32),
            pltpu.VMEM((CHUNK,), jnp.int32),
            pltpu.VMEM((1, CHUNK), jnp.int32),
            pltpu.VMEM((1, CHUNK), jnp.int32),
            pltpu.VMEM((1, CHUNK), jnp.int32),
            pltpu.VMEM((CHUNK, D), jnp.float32),
            pltpu.VMEM((CHUNK, D), jnp.float32),
            pltpu.VMEM((CHUNK, D), jnp.float32),
            pltpu.VMEM_SHARED((N, D), jnp.float32),
            pltpu.SemaphoreType.DMA,
            pltpu.SemaphoreType.DMA,
            pltpu.SemaphoreType.DMA,
            pltpu.SemaphoreType.DMA,
            pltpu.SemaphoreType.DMA,
            pltpu.SemaphoreType.DMA,
            pltpu.SemaphoreType.DMA,
            pltpu.SemaphoreType.DMA,
            pltpu.SemaphoreType.DMA,
        ],
    )
    def agg_kernel(x_hbm, src_hbm, dst_hbm, zero_hbm, out_hbm,
                   sv0, sv1, sv2, dv0, dv1, dv2, r0, r1, r2, acc,
                   g0, g1, g2, ss0, ss1, ss2, i0, i1, i2):
        srcs = (sv0, sv1, sv2)
        dsts = (dv0, dv1, dv2)
        rows = (r0, r1, r2)
        gs = (g0, g1, g2)
        ssems = (ss0, ss1, ss2)
        isems = (i0, i1, i2)
        c = lax.axis_index("c")
        sid = lax.axis_index("s")
        w = c * NS + sid
        row0 = sid * ROWS_PER_TILE

        pltpu.sync_copy(zero_hbm.at[pl.ds(row0, ROWS_PER_TILE)],
                        acc.at[pl.ds(row0, ROWS_PER_TILE)])

        @pl.when(sid == 0)
        def _():
            pltpu.sync_copy(zero_hbm.at[pl.ds(REM0, N - REM0)],
                            acc.at[pl.ds(REM0, N - REM0)])

        plsc.subcore_barrier()

        nch = CPW + jnp.where(w < EXTRA, 1, 0)
        base = CPW * w + jnp.minimum(w, EXTRA)

        off0 = base * CHUNK
        pltpu.sync_copy(src_hbm.at[pl.ds(off0, CHUNK)], sv0)
        pltpu.sync_copy(dst_hbm.at[pl.ds(off0, CHUNK)], dv0.at[0])

        @pl.when(1 < nch)
        def _():
            off1 = (base + 1) * CHUNK
            pltpu.async_copy(src_hbm.at[pl.ds(off1, CHUNK)], sv1, i1)
            pltpu.async_copy(dst_hbm.at[pl.ds(off1, CHUNK)], dv1.at[0], i1)

        pltpu.async_copy(x_hbm.at[sv0], r0, g0)

        def outer(t, carry):
            for b in range(3):
                j = 3 * t + b
                b1 = (b + 1) % 3
                b2 = (b + 2) % 3

                @pl.when(j < nch)
                def _():
                    pltpu.make_async_copy(
                        x_hbm.at[srcs[b]], rows[b], gs[b]).wait()

                    @pl.when(j + 2 < nch)
                    def _():
                        off = (base + j + 2) * CHUNK
                        pltpu.async_copy(
                            src_hbm.at[pl.ds(off, CHUNK)], srcs[b2],
                            isems[b2])
                        pltpu.async_copy(
                            dst_hbm.at[pl.ds(off, CHUNK)], dsts[b2].at[0],
                            isems[b2])

                    pltpu.async_copy(
                        rows[b], acc.at[dsts[b].at[0]], ssems[b], add=True)

                    @pl.when(j + 1 < nch)
                    def _():
                        off = (base + j + 1) * CHUNK
                        pltpu.make_async_copy(
                            src_hbm.at[pl.ds(off, CHUNK)], srcs[b1],
                            isems[b1]).wait()
                        pltpu.make_async_copy(
                            dst_hbm.at[pl.ds(off, CHUNK)], dsts[b1].at[0],
                            isems[b1]).wait()

                        @pl.when(j >= 2)
                        def _():
                            pltpu.make_async_copy(
                                rows[b1], acc.at[dsts[b1].at[0]],
                                ssems[b1]).wait()

                        pltpu.async_copy(
                            x_hbm.at[srcs[b1]], rows[b1], gs[b1])
            return carry

        lax.fori_loop(0, OUTER, outer, 0)

        for k in range(3):
            pltpu.make_async_copy(
                rows[k], acc.at[dsts[k].at[0]], ssems[k]).wait()

        plsc.subcore_barrier()

        pltpu.sync_copy(acc.at[pl.ds(row0, ROWS_PER_TILE)],
                        out_hbm.at[c, pl.ds(row0, ROWS_PER_TILE)])

        @pl.when(sid == 0)
        def _():
            pltpu.sync_copy(acc.at[pl.ds(REM0, N - REM0)],
                            out_hbm.at[c, pl.ds(REM0, N - REM0)])

    return agg_kernel(x, src, dst, zeros)


BLK = 1000


def _mlp_body(eps_ref, x_ref, p_ref, w1_ref, b1_ref, w2_ref, b2_ref, o_ref):
    agg = p_ref[0] + p_ref[1]
    out = (1.0 + eps_ref[...]) * x_ref[...] + agg
    h = jnp.dot(out, w1_ref[...], preferred_element_type=jnp.float32)
    h = jnp.maximum(h + b1_ref[...], 0.0)
    o_ref[...] = (
        jnp.dot(h, w2_ref[...], preferred_element_type=jnp.float32)
        + b2_ref[...]
    )


def _mlp(x, partials, eps, W1, b1, W2, b2):
    eps2 = eps.reshape(1, 1).astype(jnp.float32)
    return pl.pallas_call(
        _mlp_body,
        grid=(N // BLK,),
        in_specs=[
            pl.BlockSpec((1, 1), lambda i: (0, 0)),
            pl.BlockSpec((BLK, D), lambda i: (i, 0)),
            pl.BlockSpec((NC, BLK, D), lambda i: (0, i, 0)),
            pl.BlockSpec((D, D), lambda i: (0, 0)),
            pl.BlockSpec((1, D), lambda i: (0, 0)),
            pl.BlockSpec((D, D), lambda i: (0, 0)),
            pl.BlockSpec((1, D), lambda i: (0, 0)),
        ],
        out_specs=pl.BlockSpec((BLK, D), lambda i: (i, 0)),
        out_shape=jax.ShapeDtypeStruct((N, D), jnp.float32),
    )(eps2, x, partials, W1, b1.reshape(1, D), W2, b2.reshape(1, D))


@jax.jit
def kernel(x, edge_idx, eps, W1, b1, W2, b2):
    ei = edge_idx.astype(jnp.int32)
    zeros = jnp.zeros((N, D), jnp.float32)
    partials = _sc_aggregate(x, ei[0], ei[1], zeros)
    return _mlp(x, partials, eps, W1, b1, W2, b2)


# speedup vs baseline: 1.0005x; 1.0005x over previous
import functools

import jax
import jax.numpy as jnp
from jax import lax
from jax.experimental import pallas as pl
from jax.experimental.pallas import tpu as pltpu
from jax.experimental.pallas import tpu_sc as plsc

N = 10000
E = 320000
D = 128

CHUNK = 128
NUM_CHUNKS = E // CHUNK
NC = 2
NS = 16
NW = NC * NS
CPW = NUM_CHUNKS // NW
EXTRA = NUM_CHUNKS - CPW * NW
MAXC = CPW + 1
OUTER = (MAXC + 3) // 4

ROWS_PER_TILE = 624
REM0 = NS * ROWS_PER_TILE


def _sc_aggregate(x, src, dst, zeros):
    mesh = plsc.VectorSubcoreMesh(core_axis_name="c", subcore_axis_name="s")

    @functools.partial(
        pl.kernel,
        mesh=mesh,
        out_type=jax.ShapeDtypeStruct((NC, N, D), jnp.float32),
        scratch_types=[
            pltpu.VMEM((CHUNK,), jnp.int32),
            pltpu.VMEM((CHUNK,), jnp.int32),
            pltpu.VMEM((CHUNK,), jnp.int32),
            pltpu.VMEM((CHUNK,), jnp.int32),
            pltpu.VMEM((1, CHUNK), jnp.int32),
            pltpu.VMEM((1, CHUNK), jnp.int32),
            pltpu.VMEM((1, CHUNK), jnp.int32),
            pltpu.VMEM((1, CHUNK), jnp.int32),
            pltpu.VMEM((CHUNK, D), jnp.float32),
            pltpu.VMEM((CHUNK, D), jnp.float32),
            pltpu.VMEM_SHARED((N, D), jnp.float32),
            pltpu.SemaphoreType.DMA,
            pltpu.SemaphoreType.DMA,
            pltpu.SemaphoreType.DMA,
            pltpu.SemaphoreType.DMA,
            pltpu.SemaphoreType.DMA,
            pltpu.SemaphoreType.DMA,
        ],
    )
    def agg_kernel(x_hbm, src_hbm, dst_hbm, zero_hbm, out_hbm,
                   sv0, sv1, sv2, sv3, dv0, dv1, dv2, dv3, r0, r1, acc,
                   g0, g1, i0, i1, i2, i3):
        srcs = (sv0, sv1, sv2, sv3)
        dsts = (dv0, dv1, dv2, dv3)
        rows = (r0, r1)
        gs = (g0, g1)
        isems = (i0, i1, i2, i3)
        c = lax.axis_index("c")
        sid = lax.axis_index("s")
        w = c * NS + sid
        row0 = sid * ROWS_PER_TILE

        pltpu.sync_copy(zero_hbm.at[pl.ds(row0, ROWS_PER_TILE)],
                        acc.at[pl.ds(row0, ROWS_PER_TILE)])

        @pl.when(sid == 0)
        def _():
            pltpu.sync_copy(zero_hbm.at[pl.ds(REM0, N - REM0)],
                            acc.at[pl.ds(REM0, N - REM0)])

        plsc.subcore_barrier()

        nch = CPW + jnp.where(w < EXTRA, 1, 0)
        base = CPW * w + jnp.minimum(w, EXTRA)

        off0 = base * CHUNK
        pltpu.sync_copy(src_hbm.at[pl.ds(off0, CHUNK)], sv0)
        pltpu.sync_copy(dst_hbm.at[pl.ds(off0, CHUNK)], dv0.at[0])

        @pl.when(1 < nch)
        def _():
            off1 = (base + 1) * CHUNK
            pltpu.async_copy(src_hbm.at[pl.ds(off1, CHUNK)], sv1, i1)
            pltpu.async_copy(dst_hbm.at[pl.ds(off1, CHUNK)], dv1.at[0], i1)

        pltpu.async_copy(x_hbm.at[sv0], r0, g0)

        def outer(t, carry):
            for b in range(4):
                j = 4 * t + b
                rb = b % 2
                rb1 = 1 - rb
                ib1 = (b + 1) % 4
                ib2 = (b + 2) % 4

                @pl.when(j < nch)
                def _():
                    pltpu.make_async_copy(
                        x_hbm.at[srcs[b]], rows[rb], gs[rb]).wait()

                    @pl.when(j + 2 < nch)
                    def _():
                        off = (base + j + 2) * CHUNK
                        pltpu.async_copy(
                            src_hbm.at[pl.ds(off, CHUNK)], srcs[ib2],
                            isems[ib2])
                        pltpu.async_copy(
                            dst_hbm.at[pl.ds(off, CHUNK)], dsts[ib2].at[0],
                            isems[ib2])

                    @pl.when(j + 1 < nch)
                    def _():
                        off = (base + j + 1) * CHUNK
                        pltpu.make_async_copy(
                            src_hbm.at[pl.ds(off, CHUNK)], srcs[ib1],
                            isems[ib1]).wait()
                        pltpu.make_async_copy(
                            dst_hbm.at[pl.ds(off, CHUNK)], dsts[ib1].at[0],
                            isems[ib1]).wait()
                        pltpu.async_copy(
                            x_hbm.at[srcs[ib1]], rows[rb1], gs[rb1])

                    pltpu.sync_copy(rows[rb], acc.at[dsts[b].at[0]], add=True)
            return carry

        lax.fori_loop(0, OUTER, outer, 0)
        plsc.subcore_barrier()

        pltpu.sync_copy(acc.at[pl.ds(row0, ROWS_PER_TILE)],
                        out_hbm.at[c, pl.ds(row0, ROWS_PER_TILE)])

        @pl.when(sid == 0)
        def _():
            pltpu.sync_copy(acc.at[pl.ds(REM0, N - REM0)],
                            out_hbm.at[c, pl.ds(REM0, N - REM0)])

    return agg_kernel(x, src, dst, zeros)


BLK = 1000


def _mlp_body(eps_ref, x_ref, p_ref, w1_ref, b1_ref, w2_ref, b2_ref, o_ref):
    agg = p_ref[0] + p_ref[1]
    out = (1.0 + eps_ref[...]) * x_ref[...] + agg
    h = jnp.dot(out, w1_ref[...], preferred_element_type=jnp.float32)
    h = jnp.maximum(h + b1_ref[...], 0.0)
    o_ref[...] = (
        jnp.dot(h, w2_ref[...], preferred_element_type=jnp.float32)
        + b2_ref[...]
    )


def _mlp(x, partials, eps, W1, b1, W2, b2):
    eps2 = eps.reshape(1, 1).astype(jnp.float32)
    return pl.pallas_call(
        _mlp_body,
        grid=(N // BLK,),
        in_specs=[
            pl.BlockSpec((1, 1), lambda i: (0, 0)),
            pl.BlockSpec((BLK, D), lambda i: (i, 0)),
            pl.BlockSpec((NC, BLK, D), lambda i: (0, i, 0)),
            pl.BlockSpec((D, D), lambda i: (0, 0)),
            pl.BlockSpec((1, D), lambda i: (0, 0)),
            pl.BlockSpec((D, D), lambda i: (0, 0)),
            pl.BlockSpec((1, D), lambda i: (0, 0)),
        ],
        out_specs=pl.BlockSpec((BLK, D), lambda i: (i, 0)),
        out_shape=jax.ShapeDtypeStruct((N, D), jnp.float32),
    )(eps2, x, partials, W1, b1.reshape(1, D), W2, b2.reshape(1, D))


@jax.jit
def kernel(x, edge_idx, eps, W1, b1, W2, b2):
    ei = edge_idx.astype(jnp.int32)
    zeros = jnp.zeros((N, D), jnp.float32)
    partials = _sc_aggregate(x, ei[0], ei[1], zeros)
    return _mlp(x, partials, eps, W1, b1, W2, b2)


# speedup vs baseline: 1.0619x; 1.0613x over previous
import functools

import jax
import jax.numpy as jnp
from jax import lax
from jax.experimental import pallas as pl
from jax.experimental.pallas import tpu as pltpu
from jax.experimental.pallas import tpu_sc as plsc

N = 10000
E = 320000
D = 128

CHUNK = 128
NUM_CHUNKS = E // CHUNK
NC = 2
NS = 16
NW = NC * NS
CPW = NUM_CHUNKS // NW
EXTRA = NUM_CHUNKS - CPW * NW
MAXC = CPW + 1
OUTER = (MAXC + 3) // 4

ROWS_PER_TILE = 624
REM0 = NS * ROWS_PER_TILE


def _sc_aggregate(x, src, dst):
    mesh = plsc.VectorSubcoreMesh(core_axis_name="c", subcore_axis_name="s")

    @functools.partial(
        pl.kernel,
        mesh=mesh,
        out_type=jax.ShapeDtypeStruct((NC, N, D), jnp.float32),
        scratch_types=[
            pltpu.VMEM((CHUNK,), jnp.int32),
            pltpu.VMEM((CHUNK,), jnp.int32),
            pltpu.VMEM((CHUNK,), jnp.int32),
            pltpu.VMEM((CHUNK,), jnp.int32),
            pltpu.VMEM((1, CHUNK), jnp.int32),
            pltpu.VMEM((1, CHUNK), jnp.int32),
            pltpu.VMEM((1, CHUNK), jnp.int32),
            pltpu.VMEM((1, CHUNK), jnp.int32),
            pltpu.VMEM((CHUNK, D), jnp.float32),
            pltpu.VMEM((CHUNK, D), jnp.float32),
            pltpu.VMEM_SHARED((N, D), jnp.float32),
            pltpu.SemaphoreType.DMA,
            pltpu.SemaphoreType.DMA,
            pltpu.SemaphoreType.DMA,
            pltpu.SemaphoreType.DMA,
            pltpu.SemaphoreType.DMA,
            pltpu.SemaphoreType.DMA,
        ],
    )
    def agg_kernel(x_hbm, src_hbm, dst_hbm, out_hbm,
                   sv0, sv1, sv2, sv3, dv0, dv1, dv2, dv3, r0, r1, acc,
                   g0, g1, i0, i1, i2, i3):
        srcs = (sv0, sv1, sv2, sv3)
        dsts = (dv0, dv1, dv2, dv3)
        rows = (r0, r1)
        gs = (g0, g1)
        isems = (i0, i1, i2, i3)
        c = lax.axis_index("c")
        sid = lax.axis_index("s")
        w = c * NS + sid
        row0 = sid * ROWS_PER_TILE

        zv = jnp.zeros((16,), jnp.float32)

        def zfill(i, carry):
            for cc in range(8):
                r0[i, pl.ds(cc * 16, 16)] = zv
            return carry

        lax.fori_loop(0, CHUNK, zfill, 0)
        for k in range(4):
            pltpu.sync_copy(r0, acc.at[pl.ds(row0 + k * CHUNK, CHUNK)])
        pltpu.sync_copy(r0.at[pl.ds(0, ROWS_PER_TILE - 4 * CHUNK)],
                        acc.at[pl.ds(row0 + 4 * CHUNK,
                                     ROWS_PER_TILE - 4 * CHUNK)])

        @pl.when(sid == 0)
        def _():
            pltpu.sync_copy(r0.at[pl.ds(0, N - REM0)],
                            acc.at[pl.ds(REM0, N - REM0)])

        plsc.subcore_barrier()

        nch = CPW + jnp.where(w < EXTRA, 1, 0)
        base = CPW * w + jnp.minimum(w, EXTRA)

        off0 = base * CHUNK
        pltpu.sync_copy(src_hbm.at[pl.ds(off0, CHUNK)], sv0)
        pltpu.sync_copy(dst_hbm.at[pl.ds(off0, CHUNK)], dv0.at[0])

        @pl.when(1 < nch)
        def _():
            off1 = (base + 1) * CHUNK
            pltpu.async_copy(src_hbm.at[pl.ds(off1, CHUNK)], sv1, i1)
            pltpu.async_copy(dst_hbm.at[pl.ds(off1, CHUNK)], dv1.at[0], i1)

        pltpu.async_copy(x_hbm.at[sv0], r0, g0)

        def outer(t, carry):
            for b in range(4):
                j = 4 * t + b
                rb = b % 2
                rb1 = 1 - rb
                ib1 = (b + 1) % 4
                ib2 = (b + 2) % 4

                @pl.when(j < nch)
                def _():
                    pltpu.make_async_copy(
                        x_hbm.at[srcs[b]], rows[rb], gs[rb]).wait()

                    @pl.when(j + 2 < nch)
                    def _():
                        off = (base + j + 2) * CHUNK
                        pltpu.async_copy(
                            src_hbm.at[pl.ds(off, CHUNK)], srcs[ib2],
                            isems[ib2])
                        pltpu.async_copy(
                            dst_hbm.at[pl.ds(off, CHUNK)], dsts[ib2].at[0],
                            isems[ib2])

                    @pl.when(j + 1 < nch)
                    def _():
                        off = (base + j + 1) * CHUNK
                        pltpu.make_async_copy(
                            src_hbm.at[pl.ds(off, CHUNK)], srcs[ib1],
                            isems[ib1]).wait()
                        pltpu.make_async_copy(
                            dst_hbm.at[pl.ds(off, CHUNK)], dsts[ib1].at[0],
                            isems[ib1]).wait()
                        pltpu.async_copy(
                            x_hbm.at[srcs[ib1]], rows[rb1], gs[rb1])

                    pltpu.sync_copy(rows[rb], acc.at[dsts[b].at[0]], add=True)
            return carry

        lax.fori_loop(0, OUTER, outer, 0)
        plsc.subcore_barrier()

        pltpu.sync_copy(acc.at[pl.ds(row0, ROWS_PER_TILE)],
                        out_hbm.at[c, pl.ds(row0, ROWS_PER_TILE)])

        @pl.when(sid == 0)
        def _():
            pltpu.sync_copy(acc.at[pl.ds(REM0, N - REM0)],
                            out_hbm.at[c, pl.ds(REM0, N - REM0)])

    return agg_kernel(x, src, dst)


BLK = 2000


def _mlp_body(eps_ref, x_ref, p_ref, w1_ref, b1_ref, w2_ref, b2_ref, o_ref):
    agg = p_ref[0] + p_ref[1]
    out = (1.0 + eps_ref[...]) * x_ref[...] + agg
    h = jnp.dot(out, w1_ref[...], preferred_element_type=jnp.float32)
    h = jnp.maximum(h + b1_ref[...], 0.0)
    o_ref[...] = (
        jnp.dot(h, w2_ref[...], preferred_element_type=jnp.float32)
        + b2_ref[...]
    )


def _mlp(x, partials, eps, W1, b1, W2, b2):
    eps2 = eps.reshape(1, 1).astype(jnp.float32)
    return pl.pallas_call(
        _mlp_body,
        grid=(N // BLK,),
        in_specs=[
            pl.BlockSpec((1, 1), lambda i: (0, 0)),
            pl.BlockSpec((BLK, D), lambda i: (i, 0)),
            pl.BlockSpec((NC, BLK, D), lambda i: (0, i, 0)),
            pl.BlockSpec((D, D), lambda i: (0, 0)),
            pl.BlockSpec((1, D), lambda i: (0, 0)),
            pl.BlockSpec((D, D), lambda i: (0, 0)),
            pl.BlockSpec((1, D), lambda i: (0, 0)),
        ],
        out_specs=pl.BlockSpec((BLK, D), lambda i: (i, 0)),
        out_shape=jax.ShapeDtypeStruct((N, D), jnp.float32),
    )(eps2, x, partials, W1, b1.reshape(1, D), W2, b2.reshape(1, D))


@jax.jit
def kernel(x, edge_idx, eps, W1, b1, W2, b2):
    ei = edge_idx.astype(jnp.int32)
    partials = _sc_aggregate(x, ei[0], ei[1])
    return _mlp(x, partials, eps, W1, b1, W2, b2)


# speedup vs baseline: 1.1307x; 1.0648x over previous
import functools

import jax
import jax.numpy as jnp
from jax import lax
from jax.experimental import pallas as pl
from jax.experimental.pallas import tpu as pltpu
from jax.experimental.pallas import tpu_sc as plsc

N = 10000
E = 320000
D = 128

CHUNK = 128
NUM_CHUNKS = E // CHUNK
NC = 2
NS = 16
NW = NC * NS
CPW = NUM_CHUNKS // NW
EXTRA = NUM_CHUNKS - CPW * NW
MAXC = CPW + 1
OUTER = (MAXC + 3) // 4

ROWS_PER_TILE = 624
REM0 = NS * ROWS_PER_TILE


def _sc_aggregate(x, edges_flat):
    mesh = plsc.VectorSubcoreMesh(core_axis_name="c", subcore_axis_name="s")

    @functools.partial(
        pl.kernel,
        mesh=mesh,
        out_type=jax.ShapeDtypeStruct((NC, N, D), jnp.float32),
        scratch_types=[
            pltpu.VMEM((CHUNK,), jnp.int32),
            pltpu.VMEM((CHUNK,), jnp.int32),
            pltpu.VMEM((CHUNK,), jnp.int32),
            pltpu.VMEM((CHUNK,), jnp.int32),
            pltpu.VMEM((1, CHUNK), jnp.int32),
            pltpu.VMEM((1, CHUNK), jnp.int32),
            pltpu.VMEM((1, CHUNK), jnp.int32),
            pltpu.VMEM((1, CHUNK), jnp.int32),
            pltpu.VMEM((CHUNK, D), jnp.float32),
            pltpu.VMEM((CHUNK, D), jnp.float32),
            pltpu.VMEM_SHARED((N, D), jnp.float32),
            pltpu.SemaphoreType.DMA,
            pltpu.SemaphoreType.DMA,
            pltpu.SemaphoreType.DMA,
            pltpu.SemaphoreType.DMA,
            pltpu.SemaphoreType.DMA,
            pltpu.SemaphoreType.DMA,
        ],
    )
    def agg_kernel(x_hbm, e_hbm, out_hbm,
                   sv0, sv1, sv2, sv3, dv0, dv1, dv2, dv3, r0, r1, acc,
                   g0, g1, i0, i1, i2, i3):
        srcs = (sv0, sv1, sv2, sv3)
        dsts = (dv0, dv1, dv2, dv3)
        rows = (r0, r1)
        gs = (g0, g1)
        isems = (i0, i1, i2, i3)
        c = lax.axis_index("c")
        sid = lax.axis_index("s")
        w = c * NS + sid
        row0 = sid * ROWS_PER_TILE

        zv = jnp.zeros((16,), jnp.float32)

        def zfill(i, carry):
            for cc in range(8):
                r0[i, pl.ds(cc * 16, 16)] = zv
            return carry

        lax.fori_loop(0, CHUNK, zfill, 0)
        for k in range(4):
            pltpu.sync_copy(r0, acc.at[pl.ds(row0 + k * CHUNK, CHUNK)])
        pltpu.sync_copy(r0.at[pl.ds(0, ROWS_PER_TILE - 4 * CHUNK)],
                        acc.at[pl.ds(row0 + 4 * CHUNK,
                                     ROWS_PER_TILE - 4 * CHUNK)])

        @pl.when(sid == 0)
        def _():
            pltpu.sync_copy(r0.at[pl.ds(0, N - REM0)],
                            acc.at[pl.ds(REM0, N - REM0)])

        plsc.subcore_barrier()

        nch = CPW + jnp.where(w < EXTRA, 1, 0)
        base = CPW * w + jnp.minimum(w, EXTRA)

        off0 = base * CHUNK
        pltpu.sync_copy(e_hbm.at[pl.ds(off0, CHUNK)], sv0)
        pltpu.sync_copy(e_hbm.at[pl.ds(E + off0, CHUNK)], dv0.at[0])

        @pl.when(1 < nch)
        def _():
            off1 = (base + 1) * CHUNK
            pltpu.async_copy(e_hbm.at[pl.ds(off1, CHUNK)], sv1, i1)
            pltpu.async_copy(e_hbm.at[pl.ds(E + off1, CHUNK)], dv1.at[0], i1)

        pltpu.async_copy(x_hbm.at[sv0], r0, g0)

        def outer(t, carry):
            for b in range(4):
                j = 4 * t + b
                rb = b % 2
                rb1 = 1 - rb
                ib1 = (b + 1) % 4
                ib2 = (b + 2) % 4

                @pl.when(j < nch)
                def _():
                    pltpu.make_async_copy(
                        x_hbm.at[srcs[b]], rows[rb], gs[rb]).wait()

                    @pl.when(j + 2 < nch)
                    def _():
                        off = (base + j + 2) * CHUNK
                        pltpu.async_copy(
                            e_hbm.at[pl.ds(off, CHUNK)], srcs[ib2],
                            isems[ib2])
                        pltpu.async_copy(
                            e_hbm.at[pl.ds(E + off, CHUNK)], dsts[ib2].at[0],
                            isems[ib2])

                    @pl.when(j + 1 < nch)
                    def _():
                        off = (base + j + 1) * CHUNK
                        pltpu.make_async_copy(
                            e_hbm.at[pl.ds(off, CHUNK)], srcs[ib1],
                            isems[ib1]).wait()
                        pltpu.make_async_copy(
                            e_hbm.at[pl.ds(E + off, CHUNK)], dsts[ib1].at[0],
                            isems[ib1]).wait()
                        pltpu.async_copy(
                            x_hbm.at[srcs[ib1]], rows[rb1], gs[rb1])

                    pltpu.sync_copy(rows[rb], acc.at[dsts[b].at[0]], add=True)
            return carry

        lax.fori_loop(0, OUTER, outer, 0)
        plsc.subcore_barrier()

        pltpu.sync_copy(acc.at[pl.ds(row0, ROWS_PER_TILE)],
                        out_hbm.at[c, pl.ds(row0, ROWS_PER_TILE)])

        @pl.when(sid == 0)
        def _():
            pltpu.sync_copy(acc.at[pl.ds(REM0, N - REM0)],
                            out_hbm.at[c, pl.ds(REM0, N - REM0)])

    return agg_kernel(x, edges_flat)


BLK = 2000


def _mlp_body(eps_ref, x_ref, p_ref, w1_ref, b1_ref, w2_ref, b2_ref, o_ref):
    agg = p_ref[0] + p_ref[1]
    out = (1.0 + eps_ref[...]) * x_ref[...] + agg
    h = jnp.dot(out, w1_ref[...], preferred_element_type=jnp.float32)
    h = jnp.maximum(h + b1_ref[...], 0.0)
    o_ref[...] = (
        jnp.dot(h, w2_ref[...], preferred_element_type=jnp.float32)
        + b2_ref[...]
    )


def _mlp(x, partials, eps, W1, b1, W2, b2):
    eps2 = eps.reshape(1, 1).astype(jnp.float32)
    return pl.pallas_call(
        _mlp_body,
        grid=(N // BLK,),
        in_specs=[
            pl.BlockSpec((1, 1), lambda i: (0, 0)),
            pl.BlockSpec((BLK, D), lambda i: (i, 0)),
            pl.BlockSpec((NC, BLK, D), lambda i: (0, i, 0)),
            pl.BlockSpec((D, D), lambda i: (0, 0)),
            pl.BlockSpec((1, D), lambda i: (0, 0)),
            pl.BlockSpec((D, D), lambda i: (0, 0)),
            pl.BlockSpec((1, D), lambda i: (0, 0)),
        ],
        out_specs=pl.BlockSpec((BLK, D), lambda i: (i, 0)),
        out_shape=jax.ShapeDtypeStruct((N, D), jnp.float32),
    )(eps2, x, partials, W1, b1.reshape(1, D), W2, b2.reshape(1, D))


@jax.jit
def kernel(x, edge_idx, eps, W1, b1, W2, b2):
    efl = edge_idx.astype(jnp.int32).reshape(-1)
    partials = _sc_aggregate(x, efl)
    return _mlp(x, partials, eps, W1, b1, W2, b2)


# speedup vs baseline: 1.1331x; 1.0021x over previous
import functools

import jax
import jax.numpy as jnp
from jax import lax
from jax.experimental import pallas as pl
from jax.experimental.pallas import tpu as pltpu
from jax.experimental.pallas import tpu_sc as plsc

N = 10000
E = 320000
D = 128

CHUNK = 128
NUM_CHUNKS = E // CHUNK
NC = 2
NS = 16
NW = NC * NS
CPW = NUM_CHUNKS // NW
EXTRA = NUM_CHUNKS - CPW * NW
MAXC = CPW + 1
OUTER = (MAXC + 3) // 4

ROWS_PER_TILE = 624
REM0 = NS * ROWS_PER_TILE


def _sc_aggregate(x, edges_flat):
    mesh = plsc.VectorSubcoreMesh(core_axis_name="c", subcore_axis_name="s")

    @functools.partial(
        pl.kernel,
        mesh=mesh,
        out_type=jax.ShapeDtypeStruct((NC, N, D), jnp.float32),
        scratch_types=[
            pltpu.VMEM((CHUNK,), jnp.int32),
            pltpu.VMEM((CHUNK,), jnp.int32),
            pltpu.VMEM((CHUNK,), jnp.int32),
            pltpu.VMEM((CHUNK,), jnp.int32),
            pltpu.VMEM((1, CHUNK), jnp.int32),
            pltpu.VMEM((1, CHUNK), jnp.int32),
            pltpu.VMEM((1, CHUNK), jnp.int32),
            pltpu.VMEM((1, CHUNK), jnp.int32),
            pltpu.VMEM((CHUNK, D), jnp.float32),
            pltpu.VMEM((CHUNK, D), jnp.float32),
            pltpu.VMEM_SHARED((N, D), jnp.float32),
            pltpu.SemaphoreType.DMA,
            pltpu.SemaphoreType.DMA,
            pltpu.SemaphoreType.DMA,
            pltpu.SemaphoreType.DMA,
            pltpu.SemaphoreType.DMA,
            pltpu.SemaphoreType.DMA,
        ],
    )
    def agg_kernel(x_hbm, e_hbm, out_hbm,
                   sv0, sv1, sv2, sv3, dv0, dv1, dv2, dv3, r0, r1, acc,
                   g0, g1, i0, i1, i2, i3):
        srcs = (sv0, sv1, sv2, sv3)
        dsts = (dv0, dv1, dv2, dv3)
        rows = (r0, r1)
        gs = (g0, g1)
        isems = (i0, i1, i2, i3)
        c = lax.axis_index("c")
        sid = lax.axis_index("s")
        w = c * NS + sid
        row0 = sid * ROWS_PER_TILE

        nch = CPW + jnp.where(w < EXTRA, 1, 0)
        base = CPW * w + jnp.minimum(w, EXTRA)
        off0 = base * CHUNK

        pltpu.async_copy(e_hbm.at[pl.ds(off0, CHUNK)], sv0, i0)
        pltpu.async_copy(e_hbm.at[pl.ds(E + off0, CHUNK)], dv0.at[0], i0)

        @pl.when(1 < nch)
        def _():
            off1 = (base + 1) * CHUNK
            pltpu.async_copy(e_hbm.at[pl.ds(off1, CHUNK)], sv1, i1)
            pltpu.async_copy(e_hbm.at[pl.ds(E + off1, CHUNK)], dv1.at[0], i1)

        zv = jnp.zeros((16,), jnp.float32)

        def zfill(i, carry):
            for cc in range(8):
                r0[i, pl.ds(cc * 16, 16)] = zv
            return carry

        lax.fori_loop(0, CHUNK, zfill, 0)
        for k in range(4):
            pltpu.sync_copy(r0, acc.at[pl.ds(row0 + k * CHUNK, CHUNK)])
        pltpu.sync_copy(r0.at[pl.ds(0, ROWS_PER_TILE - 4 * CHUNK)],
                        acc.at[pl.ds(row0 + 4 * CHUNK,
                                     ROWS_PER_TILE - 4 * CHUNK)])

        @pl.when(sid == 0)
        def _():
            pltpu.sync_copy(r0.at[pl.ds(0, N - REM0)],
                            acc.at[pl.ds(REM0, N - REM0)])

        pltpu.make_async_copy(
            e_hbm.at[pl.ds(off0, CHUNK)], sv0, i0).wait()
        pltpu.make_async_copy(
            e_hbm.at[pl.ds(E + off0, CHUNK)], dv0.at[0], i0).wait()
        pltpu.async_copy(x_hbm.at[sv0], r0, g0)

        plsc.subcore_barrier()

        def outer(t, carry):
            for b in range(4):
                j = 4 * t + b
                rb = b % 2
                rb1 = 1 - rb
                ib1 = (b + 1) % 4
                ib2 = (b + 2) % 4

                @pl.when(j < nch)
                def _():
                    pltpu.make_async_copy(
                        x_hbm.at[srcs[b]], rows[rb], gs[rb]).wait()

                    @pl.when(j + 2 < nch)
                    def _():
                        off = (base + j + 2) * CHUNK
                        pltpu.async_copy(
                            e_hbm.at[pl.ds(off, CHUNK)], srcs[ib2],
                            isems[ib2])
                        pltpu.async_copy(
                            e_hbm.at[pl.ds(E + off, CHUNK)], dsts[ib2].at[0],
                            isems[ib2])

                    @pl.when(j + 1 < nch)
                    def _():
                        off = (base + j + 1) * CHUNK
                        pltpu.make_async_copy(
                            e_hbm.at[pl.ds(off, CHUNK)], srcs[ib1],
                            isems[ib1]).wait()
                        pltpu.make_async_copy(
                            e_hbm.at[pl.ds(E + off, CHUNK)], dsts[ib1].at[0],
                            isems[ib1]).wait()
                        pltpu.async_copy(
                            x_hbm.at[srcs[ib1]], rows[rb1], gs[rb1])

                    pltpu.sync_copy(rows[rb], acc.at[dsts[b].at[0]], add=True)
            return carry

        lax.fori_loop(0, OUTER, outer, 0)
        plsc.subcore_barrier()

        pltpu.sync_copy(acc.at[pl.ds(row0, ROWS_PER_TILE)],
                        out_hbm.at[c, pl.ds(row0, ROWS_PER_TILE)])

        @pl.when(sid == 0)
        def _():
            pltpu.sync_copy(acc.at[pl.ds(REM0, N - REM0)],
                            out_hbm.at[c, pl.ds(REM0, N - REM0)])

    return agg_kernel(x, edges_flat)


BLK = 2000


def _mlp_body(eps_ref, x_ref, p_ref, w1_ref, b1_ref, w2_ref, b2_ref, o_ref):
    agg = p_ref[0] + p_ref[1]
    out = (1.0 + eps_ref[...]) * x_ref[...] + agg
    h = jnp.dot(out, w1_ref[...], preferred_element_type=jnp.float32)
    h = jnp.maximum(h + b1_ref[...], 0.0)
    o_ref[...] = (
        jnp.dot(h, w2_ref[...], preferred_element_type=jnp.float32)
        + b2_ref[...]
    )


def _mlp(x, partials, eps, W1, b1, W2, b2):
    eps2 = eps.reshape(1, 1).astype(jnp.float32)
    return pl.pallas_call(
        _mlp_body,
        grid=(N // BLK,),
        in_specs=[
            pl.BlockSpec((1, 1), lambda i: (0, 0)),
            pl.BlockSpec((BLK, D), lambda i: (i, 0)),
            pl.BlockSpec((NC, BLK, D), lambda i: (0, i, 0)),
            pl.BlockSpec((D, D), lambda i: (0, 0)),
            pl.BlockSpec((1, D), lambda i: (0, 0)),
            pl.BlockSpec((D, D), lambda i: (0, 0)),
            pl.BlockSpec((1, D), lambda i: (0, 0)),
        ],
        out_specs=pl.BlockSpec((BLK, D), lambda i: (i, 0)),
        out_shape=jax.ShapeDtypeStruct((N, D), jnp.float32),
    )(eps2, x, partials, W1, b1.reshape(1, D), W2, b2.reshape(1, D))


@jax.jit
def kernel(x, edge_idx, eps, W1, b1, W2, b2):
    efl = edge_idx.astype(jnp.int32).reshape(-1)
    partials = _sc_aggregate(x, efl)
    return _mlp(x, partials, eps, W1, b1, W2, b2)
